# Initial kernel scaffold; baseline (speedup 1.0000x reference)
import jax
import jax.numpy as jnp
from jax.experimental import pallas as pl


def kernel(x, W_in, b_in, g_in, be_in, Wl_t, bl_t, Wr_t, Wl_c, bl_c, Wr_c, g_ln, b_ln, Wg1, bg1, Wg2, bg2, Wd1, bd1, Wd2, bd2, ei_t, ei_c):
    raise NotImplementedError("write your pallas kernel here")


# speedup vs baseline: 2.7524x; 2.7524x over previous
import jax
import jax.numpy as jnp
from jax import lax
from jax.experimental import pallas as pl
from jax.experimental.pallas import tpu as pltpu
from jax.experimental.pallas import tpu_sc as plsc

N = 10000
E = 160000
D = 256
HD = 128
NG = 5
L = 3

NS = 16
K = 128
S = 20
NSTG = 4
EPT = K * S * NSTG
EP = NS * EPT
NB = 2
NP = 10112
RPT = NP // NS

RB = 400
GRID = N // RB


def _agg_body(h_lo, h_hi, src4_t, dst4_t, src4_c, dst4_c, zeros_h,
              st_lo, st_hi, sc_lo, sc_hi,
              acc, srcb, dstb, b0, b1, s0, s1):
    cid = lax.axis_index("c")
    sid = lax.axis_index("s")
    bufs = (b0, b1)
    sems = (s0, s1)

    def one_relation(h_half, src4, dst4, out_half):
        pltpu.sync_copy(zeros_h, acc.at[pl.ds(sid * RPT, RPT)])
        plsc.subcore_barrier()

        for st in range(NSTG):
            pltpu.sync_copy(src4.at[sid, st], srcb)
            pltpu.sync_copy(dst4.at[sid, st], dstb)
            for b in range(NB):
                pltpu.async_copy(h_half.at[srcb.at[b]], bufs[b], sems[b])

            def grp(g, carry):
                for b in range(NB):
                    jj = g * NB + b
                    pltpu.make_async_copy(
                        h_half.at[srcb.at[jj]], bufs[b], sems[b]).wait()
                    pltpu.sync_copy(bufs[b], acc.at[dstb.at[jj]], add=True)
                    pltpu.async_copy(
                        h_half.at[srcb.at[jj + NB]], bufs[b], sems[b])
                return carry

            lax.fori_loop(0, (S - NB) // NB, grp, 0)
            for b in range(NB):
                jj = S - NB + b
                pltpu.make_async_copy(
                    h_half.at[srcb.at[jj]], bufs[b], sems[b]).wait()
                pltpu.sync_copy(bufs[b], acc.at[dstb.at[jj]], add=True)

        plsc.subcore_barrier()
        pltpu.sync_copy(acc.at[pl.ds(sid * RPT, RPT)],
                        out_half.at[pl.ds(sid * RPT, RPT)])
        plsc.subcore_barrier()

    @pl.when(cid == 0)
    def _():
        one_relation(h_lo, src4_t, dst4_t, st_lo)
        one_relation(h_lo, src4_c, dst4_c, sc_lo)

    @pl.when(cid == 1)
    def _():
        one_relation(h_hi, src4_t, dst4_t, st_hi)
        one_relation(h_hi, src4_c, dst4_c, sc_hi)


_agg_call = pl.kernel(
    _agg_body,
    out_type=[jax.ShapeDtypeStruct((NP, HD), jnp.float32)] * 4,
    mesh=plsc.VectorSubcoreMesh(core_axis_name="c", subcore_axis_name="s"),
    scratch_types=(
        [pltpu.VMEM_SHARED((NP, HD), jnp.float32),
         pltpu.VMEM((S, K), jnp.int32),
         pltpu.VMEM((S, K), jnp.int32)]
        + [pltpu.VMEM((K, HD), jnp.float32) for _ in range(NB)]
        + [pltpu.SemaphoreType.DMA for _ in range(NB)]
    ),
)


def _cnt_body(dst4_t, dst4_c, zeros_h, ones_h, cnt_t, cnt_c,
              acc, dstb, ones_v, sem):
    cid = lax.axis_index("c")
    sid = lax.axis_index("s")

    def run(dst4, out):
        pltpu.sync_copy(zeros_h, acc.at[pl.ds(sid * RPT, RPT)])
        pltpu.sync_copy(ones_h, ones_v)
        plsc.subcore_barrier()

        for st in range(NSTG):
            pltpu.sync_copy(dst4.at[sid, st], dstb)

            def grp(g, carry):
                for b in range(NB):
                    pltpu.async_copy(
                        ones_v, acc.at[dstb.at[g * NB + b]], sem, add=True)
                for b in range(NB):
                    pltpu.make_async_copy(
                        ones_v, acc.at[dstb.at[g * NB + b]], sem).wait()
                return carry

            lax.fori_loop(0, S // NB, grp, 0)

        plsc.subcore_barrier()
        pltpu.sync_copy(acc.at[pl.ds(sid * RPT, RPT)],
                        out.at[pl.ds(sid * RPT, RPT)])

    @pl.when(cid == 0)
    def _():
        run(dst4_t, cnt_t)

    @pl.when(cid == 1)
    def _():
        run(dst4_c, cnt_c)


_cnt_call = pl.kernel(
    _cnt_body,
    out_type=[jax.ShapeDtypeStruct((NP, HD), jnp.float32)] * 2,
    mesh=plsc.VectorSubcoreMesh(core_axis_name="c", subcore_axis_name="s"),
    scratch_types=[
        pltpu.VMEM_SHARED((NP, HD), jnp.float32),
        pltpu.VMEM((S, K), jnp.int32),
        pltpu.VMEM((K, HD), jnp.float32),
        pltpu.SemaphoreType.DMA,
    ],
)


def _ln_rows(o, g, b):
    m = jnp.mean(o, axis=-1, keepdims=True)
    v = jnp.mean((o - m) * (o - m), axis=-1, keepdims=True)
    return (o - m) * lax.rsqrt(v + 1e-5) * g + b


def _in_body(x_ref, w_ref, b_ref, g_ref, be_ref, lo_ref, hi_ref):
    o = jnp.dot(x_ref[...], w_ref[...],
                preferred_element_type=jnp.float32) + b_ref[...]
    h = jnp.maximum(_ln_rows(o, g_ref[...], be_ref[...]), 0.0)
    lo_ref[...] = h[:, :HD]
    hi_ref[...] = h[:, HD:]


_in_call = pl.pallas_call(
    _in_body,
    grid=(GRID,),
    in_specs=[
        pl.BlockSpec((RB, D), lambda i: (i, 0)),
        pl.BlockSpec((D, D), lambda i: (0, 0)),
        pl.BlockSpec((1, D), lambda i: (0, 0)),
        pl.BlockSpec((1, D), lambda i: (0, 0)),
        pl.BlockSpec((1, D), lambda i: (0, 0)),
    ],
    out_specs=[pl.BlockSpec((RB, HD), lambda i: (i, 0))] * 2,
    out_shape=[jax.ShapeDtypeStruct((N, HD), jnp.float32)] * 2,
    compiler_params=pltpu.CompilerParams(
        dimension_semantics=("parallel",)),
)


def _comb_body(hlo, hhi, stlo, sthi, sclo, schi, ct, cc,
               wlt, wlc, wrt, wrc, blt, blc, g, b, olo, ohi):
    rt = 1.0 / jnp.maximum(ct[...], 1.0)
    rc = 1.0 / jnp.maximum(cc[...], 1.0)
    at = jnp.concatenate([stlo[...], sthi[...]], axis=1) * rt
    ac = jnp.concatenate([sclo[...], schi[...]], axis=1) * rc
    h = jnp.concatenate([hlo[...], hhi[...]], axis=1)
    o = (jnp.dot(at, wlt[...], preferred_element_type=jnp.float32)
         + jnp.dot(ac, wlc[...], preferred_element_type=jnp.float32)
         + jnp.dot(h, wrt[...] + wrc[...],
                   preferred_element_type=jnp.float32)
         + blt[...] + blc[...] + h)
    hn = _ln_rows(o, g[...], b[...])
    olo[...] = hn[:, :HD]
    ohi[...] = hn[:, HD:]


_comb_call = pl.pallas_call(
    _comb_body,
    grid=(GRID,),
    in_specs=(
        [pl.BlockSpec((RB, HD), lambda i: (i, 0))] * 6
        + [pl.BlockSpec((RB, 1), lambda i: (i, 0))] * 2
        + [pl.BlockSpec((D, D), lambda i: (0, 0))] * 4
        + [pl.BlockSpec((1, D), lambda i: (0, 0))] * 4
    ),
    out_specs=[pl.BlockSpec((RB, HD), lambda i: (i, 0))] * 2,
    out_shape=[jax.ShapeDtypeStruct((N, HD), jnp.float32)] * 2,
    compiler_params=pltpu.CompilerParams(
        dimension_semantics=("parallel",)),
)


def _heads_body(hlo, hhi, wg1, bg1, wg2, bg2, wd1, bd1, wd2, bd2,
                gate_ref, delay_ref):
    h = jnp.concatenate([hlo[...], hhi[...]], axis=1)
    tg = jnp.maximum(
        jnp.dot(h, wg1[...], preferred_element_type=jnp.float32)
        + bg1[...], 0.0)
    gate_ref[...] = jnp.dot(
        tg, wg2[...], preferred_element_type=jnp.float32) + bg2[...]
    td = jnp.maximum(
        jnp.dot(h, wd1[...], preferred_element_type=jnp.float32)
        + bd1[...], 0.0)
    delay_ref[...] = jnp.dot(
        td, wd2[...], preferred_element_type=jnp.float32) + bd2[...]


_heads_call = pl.pallas_call(
    _heads_body,
    grid=(GRID,),
    in_specs=(
        [pl.BlockSpec((RB, HD), lambda i: (i, 0))] * 2
        + [pl.BlockSpec((D, HD), lambda i: (0, 0)),
           pl.BlockSpec((1, HD), lambda i: (0, 0)),
           pl.BlockSpec((HD, HD), lambda i: (0, 0)),
           pl.BlockSpec((1, HD), lambda i: (0, 0)),
           pl.BlockSpec((D, HD), lambda i: (0, 0)),
           pl.BlockSpec((1, HD), lambda i: (0, 0)),
           pl.BlockSpec((HD, HD), lambda i: (0, 0)),
           pl.BlockSpec((1, HD), lambda i: (0, 0))]
    ),
    out_specs=[pl.BlockSpec((RB, HD), lambda i: (i, 0))] * 2,
    out_shape=[jax.ShapeDtypeStruct((N, HD), jnp.float32)] * 2,
    compiler_params=pltpu.CompilerParams(
        dimension_semantics=("parallel",)),
)


def _edge_views(ei):
    ei = ei.astype(jnp.int32)
    src = jnp.concatenate([ei[0], jnp.zeros((EP - E,), jnp.int32)])
    dst = jnp.concatenate([ei[1], jnp.full((EP - E,), N, jnp.int32)])
    return (src.reshape(NS, NSTG, S, K), dst.reshape(NS, NSTG, S, K))


def kernel(x, W_in, b_in, g_in, be_in, Wl_t, bl_t, Wr_t, Wl_c, bl_c, Wr_c,
           g_ln, b_ln, Wg1, bg1, Wg2, bg2, Wd1, bd1, Wd2, bd2, ei_t, ei_c):
    f32 = jnp.float32
    src4_t, dst4_t = _edge_views(ei_t)
    src4_c, dst4_c = _edge_views(ei_c)
    zeros_h = jnp.zeros((RPT, HD), f32)
    ones_h = jnp.ones((K, HD), f32)

    h_lo, h_hi = _in_call(x, W_in, b_in.reshape(1, D), g_in.reshape(1, D),
                          be_in.reshape(1, D))
    cnt_t_full, cnt_c_full = _cnt_call(dst4_t, dst4_c, zeros_h, ones_h)
    cnt_t = cnt_t_full[:, :1]
    cnt_c = cnt_c_full[:, :1]

    for l in range(L):
        st_lo, st_hi, sc_lo, sc_hi = _agg_call(
            h_lo, h_hi, src4_t, dst4_t, src4_c, dst4_c, zeros_h)
        h_lo, h_hi = _comb_call(
            h_lo, h_hi, st_lo, st_hi, sc_lo, sc_hi, cnt_t, cnt_c,
            Wl_t[l], Wl_c[l], Wr_t[l], Wr_c[l],
            bl_t[l].reshape(1, D), bl_c[l].reshape(1, D),
            g_ln[l].reshape(1, D), b_ln[l].reshape(1, D))

    wg2p = jnp.pad(Wg2, ((0, 0), (0, HD - NG)))
    bg2p = jnp.pad(bg2, (0, HD - NG)).reshape(1, HD)
    wd2p = jnp.pad(Wd2, ((0, 0), (0, HD - 1)))
    bd2p = jnp.pad(bd2, (0, HD - 1)).reshape(1, HD)
    gate_p, delay_p = _heads_call(
        h_lo, h_hi, Wg1, bg1.reshape(1, HD), wg2p, bg2p,
        Wd1, bd1.reshape(1, HD), wd2p, bd2p)
    return gate_p[:, :NG], delay_p[:, 0]


# speedup vs baseline: 5.6346x; 2.0472x over previous
import jax
import jax.numpy as jnp
from jax import lax
from jax.experimental import pallas as pl
from jax.experimental.pallas import tpu as pltpu
from jax.experimental.pallas import tpu_sc as plsc

N = 10000
E = 160000
D = 256
HD = 128
NG = 5
L = 3

NS = 16
K = 128
S = 20
NSTG = 4
EPT = K * S * NSTG
EP = NS * EPT
NB = 2
NP = 10112
RPT = NP // NS

RB = 400
GRID = N // RB


def _agg_body(h_lo, h_hi, src4_t, dst4_t, src4_c, dst4_c, zeros_h,
              st_lo, st_hi, sc_lo, sc_hi,
              acc, srcb, dstb, b0, b1, s0, s1):
    cid = lax.axis_index("c")
    sid = lax.axis_index("s")
    bufs = (b0, b1)
    sems = (s0, s1)

    def one_relation(h_half, src4, dst4, out_half):
        pltpu.sync_copy(zeros_h, acc.at[pl.ds(sid * RPT, RPT)])
        plsc.subcore_barrier()

        for st in range(NSTG):
            pltpu.sync_copy(src4.at[sid, st], srcb)
            pltpu.sync_copy(dst4.at[sid, st], dstb)
            for b in range(NB):
                pltpu.async_copy(h_half.at[srcb.at[b]], bufs[b], sems[b])

            def grp(g, carry):
                for b in range(NB):
                    jj = g * NB + b
                    pltpu.make_async_copy(
                        h_half.at[srcb.at[jj]], bufs[b], sems[b]).wait()
                    pltpu.sync_copy(bufs[b], acc.at[dstb.at[jj]], add=True)
                    pltpu.async_copy(
                        h_half.at[srcb.at[jj + NB]], bufs[b], sems[b])
                return carry

            lax.fori_loop(0, (S - NB) // NB, grp, 0)
            for b in range(NB):
                jj = S - NB + b
                pltpu.make_async_copy(
                    h_half.at[srcb.at[jj]], bufs[b], sems[b]).wait()
                pltpu.sync_copy(bufs[b], acc.at[dstb.at[jj]], add=True)

        plsc.subcore_barrier()
        pltpu.sync_copy(acc.at[pl.ds(sid * RPT, RPT)],
                        out_half.at[pl.ds(sid * RPT, RPT)])
        plsc.subcore_barrier()

    @pl.when(cid == 0)
    def _():
        one_relation(h_lo, src4_t, dst4_t, st_lo)
        one_relation(h_lo, src4_c, dst4_c, sc_lo)

    @pl.when(cid == 1)
    def _():
        one_relation(h_hi, src4_t, dst4_t, st_hi)
        one_relation(h_hi, src4_c, dst4_c, sc_hi)


_agg_call = pl.kernel(
    _agg_body,
    out_type=[jax.ShapeDtypeStruct((NP, HD), jnp.float32)] * 4,
    mesh=plsc.VectorSubcoreMesh(core_axis_name="c", subcore_axis_name="s"),
    scratch_types=(
        [pltpu.VMEM_SHARED((NP, HD), jnp.float32),
         pltpu.VMEM((S, K), jnp.int32),
         pltpu.VMEM((S, K), jnp.int32)]
        + [pltpu.VMEM((K, HD), jnp.float32) for _ in range(NB)]
        + [pltpu.SemaphoreType.DMA for _ in range(NB)]
    ),
)


def _cnt_body(dst4_t, dst4_c, zeros_h, ones_h, cnt_t, cnt_c,
              acc, dstb, ones_v, sem):
    cid = lax.axis_index("c")
    sid = lax.axis_index("s")

    def run(dst4, out):
        pltpu.sync_copy(zeros_h, acc.at[pl.ds(sid * RPT, RPT)])
        pltpu.sync_copy(ones_h, ones_v)
        plsc.subcore_barrier()

        for st in range(NSTG):
            pltpu.sync_copy(dst4.at[sid, st], dstb)

            def grp(g, carry):
                for b in range(NB):
                    pltpu.async_copy(
                        ones_v, acc.at[dstb.at[g * NB + b]], sem, add=True)
                for b in range(NB):
                    pltpu.make_async_copy(
                        ones_v, acc.at[dstb.at[g * NB + b]], sem).wait()
                return carry

            lax.fori_loop(0, S // NB, grp, 0)

        plsc.subcore_barrier()
        pltpu.sync_copy(acc.at[pl.ds(sid * RPT, RPT)],
                        out.at[pl.ds(sid * RPT, RPT)])

    @pl.when(cid == 0)
    def _():
        run(dst4_t, cnt_t)

    @pl.when(cid == 1)
    def _():
        run(dst4_c, cnt_c)


_cnt_call = pl.kernel(
    _cnt_body,
    out_type=[jax.ShapeDtypeStruct((NP, HD), jnp.float32)] * 2,
    mesh=plsc.VectorSubcoreMesh(core_axis_name="c", subcore_axis_name="s"),
    scratch_types=[
        pltpu.VMEM_SHARED((NP, HD), jnp.float32),
        pltpu.VMEM((S, K), jnp.int32),
        pltpu.VMEM((K, HD), jnp.float32),
        pltpu.SemaphoreType.DMA,
    ],
)


def _ln_rows(o, g, b):
    m = jnp.mean(o, axis=-1, keepdims=True)
    v = jnp.mean((o - m) * (o - m), axis=-1, keepdims=True)
    return (o - m) * lax.rsqrt(v + 1e-5) * g + b


def _in_body(x_ref, w_ref, b_ref, g_ref, be_ref, lo_ref, hi_ref):
    o = jnp.dot(x_ref[...], w_ref[...],
                preferred_element_type=jnp.float32) + b_ref[...]
    h = jnp.maximum(_ln_rows(o, g_ref[...], be_ref[...]), 0.0)
    lo_ref[...] = h[:, :HD]
    hi_ref[...] = h[:, HD:]


_in_call = pl.pallas_call(
    _in_body,
    grid=(GRID,),
    in_specs=[
        pl.BlockSpec((RB, D), lambda i: (i, 0)),
        pl.BlockSpec((D, D), lambda i: (0, 0)),
        pl.BlockSpec((1, D), lambda i: (0, 0)),
        pl.BlockSpec((1, D), lambda i: (0, 0)),
        pl.BlockSpec((1, D), lambda i: (0, 0)),
    ],
    out_specs=[pl.BlockSpec((RB, HD), lambda i: (i, 0))] * 2,
    out_shape=[jax.ShapeDtypeStruct((N, HD), jnp.float32)] * 2,
    compiler_params=pltpu.CompilerParams(
        dimension_semantics=("parallel",)),
)


def _comb_body(hlo, hhi, stlo, sthi, sclo, schi, ct, cc,
               wlt, wlc, wrt, wrc, blt, blc, g, b, olo, ohi):
    rt = 1.0 / jnp.maximum(ct[...], 1.0)
    rc = 1.0 / jnp.maximum(cc[...], 1.0)
    at = jnp.concatenate([stlo[...], sthi[...]], axis=1) * rt
    ac = jnp.concatenate([sclo[...], schi[...]], axis=1) * rc
    h = jnp.concatenate([hlo[...], hhi[...]], axis=1)
    o = (jnp.dot(at, wlt[...], preferred_element_type=jnp.float32)
         + jnp.dot(ac, wlc[...], preferred_element_type=jnp.float32)
         + jnp.dot(h, wrt[...] + wrc[...],
                   preferred_element_type=jnp.float32)
         + blt[...] + blc[...] + h)
    hn = _ln_rows(o, g[...], b[...])
    olo[...] = hn[:, :HD]
    ohi[...] = hn[:, HD:]


_comb_call = pl.pallas_call(
    _comb_body,
    grid=(GRID,),
    in_specs=(
        [pl.BlockSpec((RB, HD), lambda i: (i, 0))] * 6
        + [pl.BlockSpec((RB, 1), lambda i: (i, 0))] * 2
        + [pl.BlockSpec((D, D), lambda i: (0, 0))] * 4
        + [pl.BlockSpec((1, D), lambda i: (0, 0))] * 4
    ),
    out_specs=[pl.BlockSpec((RB, HD), lambda i: (i, 0))] * 2,
    out_shape=[jax.ShapeDtypeStruct((N, HD), jnp.float32)] * 2,
    compiler_params=pltpu.CompilerParams(
        dimension_semantics=("parallel",)),
)


def _heads_body(hlo, hhi, wg1, bg1, wg2, bg2, wd1, bd1, wd2, bd2,
                gate_ref, delay_ref):
    h = jnp.concatenate([hlo[...], hhi[...]], axis=1)
    tg = jnp.maximum(
        jnp.dot(h, wg1[...], preferred_element_type=jnp.float32)
        + bg1[...], 0.0)
    gate_ref[...] = jnp.dot(
        tg, wg2[...], preferred_element_type=jnp.float32) + bg2[...]
    td = jnp.maximum(
        jnp.dot(h, wd1[...], preferred_element_type=jnp.float32)
        + bd1[...], 0.0)
    delay_ref[...] = jnp.dot(
        td, wd2[...], preferred_element_type=jnp.float32) + bd2[...]


_heads_call = pl.pallas_call(
    _heads_body,
    grid=(GRID,),
    in_specs=(
        [pl.BlockSpec((RB, HD), lambda i: (i, 0))] * 2
        + [pl.BlockSpec((D, HD), lambda i: (0, 0)),
           pl.BlockSpec((1, HD), lambda i: (0, 0)),
           pl.BlockSpec((HD, HD), lambda i: (0, 0)),
           pl.BlockSpec((1, HD), lambda i: (0, 0)),
           pl.BlockSpec((D, HD), lambda i: (0, 0)),
           pl.BlockSpec((1, HD), lambda i: (0, 0)),
           pl.BlockSpec((HD, HD), lambda i: (0, 0)),
           pl.BlockSpec((1, HD), lambda i: (0, 0))]
    ),
    out_specs=[pl.BlockSpec((RB, HD), lambda i: (i, 0))] * 2,
    out_shape=[jax.ShapeDtypeStruct((N, HD), jnp.float32)] * 2,
    compiler_params=pltpu.CompilerParams(
        dimension_semantics=("parallel",)),
)


def _edge_views(ei):
    ei = ei.astype(jnp.int32)
    pad = jnp.arange(EP - E, dtype=jnp.int32)
    src = jnp.concatenate([ei[0], pad % N])
    dst = jnp.concatenate([ei[1], N + pad % (NP - N)])
    return (src.reshape(NS, NSTG, S, K), dst.reshape(NS, NSTG, S, K))


def kernel(x, W_in, b_in, g_in, be_in, Wl_t, bl_t, Wr_t, Wl_c, bl_c, Wr_c,
           g_ln, b_ln, Wg1, bg1, Wg2, bg2, Wd1, bd1, Wd2, bd2, ei_t, ei_c):
    f32 = jnp.float32
    src4_t, dst4_t = _edge_views(ei_t)
    src4_c, dst4_c = _edge_views(ei_c)
    zeros_h = jnp.zeros((RPT, HD), f32)
    ones_h = jnp.ones((K, HD), f32)

    h_lo, h_hi = _in_call(x, W_in, b_in.reshape(1, D), g_in.reshape(1, D),
                          be_in.reshape(1, D))
    cnt_t_full, cnt_c_full = _cnt_call(dst4_t, dst4_c, zeros_h, ones_h)
    cnt_t = cnt_t_full[:, :1]
    cnt_c = cnt_c_full[:, :1]

    for l in range(L):
        st_lo, st_hi, sc_lo, sc_hi = _agg_call(
            h_lo, h_hi, src4_t, dst4_t, src4_c, dst4_c, zeros_h)
        h_lo, h_hi = _comb_call(
            h_lo, h_hi, st_lo, st_hi, sc_lo, sc_hi, cnt_t, cnt_c,
            Wl_t[l], Wl_c[l], Wr_t[l], Wr_c[l],
            bl_t[l].reshape(1, D), bl_c[l].reshape(1, D),
            g_ln[l].reshape(1, D), b_ln[l].reshape(1, D))

    wg2p = jnp.pad(Wg2, ((0, 0), (0, HD - NG)))
    bg2p = jnp.pad(bg2, (0, HD - NG)).reshape(1, HD)
    wd2p = jnp.pad(Wd2, ((0, 0), (0, HD - 1)))
    bd2p = jnp.pad(bd2, (0, HD - 1)).reshape(1, HD)
    gate_p, delay_p = _heads_call(
        h_lo, h_hi, Wg1, bg1.reshape(1, HD), wg2p, bg2p,
        Wd1, bd1.reshape(1, HD), wd2p, bd2p)
    return gate_p[:, :NG], delay_p[:, 0]


# speedup vs baseline: 5.7487x; 1.0202x over previous
import jax
import jax.numpy as jnp
from jax import lax
from jax.experimental import pallas as pl
from jax.experimental.pallas import tpu as pltpu
from jax.experimental.pallas import tpu_sc as plsc

N = 10000
E = 160000
D = 256
HD = 128
NG = 5
L = 3

NS = 16
K = 128
S = 20
NSTG = 4
EPT = K * S * NSTG
EP = NS * EPT
NB = 2
NP = 10112
RPT = NP // NS

RB = 400
GRID = N // RB


def _agg_body(h_lo, h_hi, src4_t, dst4_t, src4_c, dst4_c, zeros_h,
              st_lo, st_hi, sc_lo, sc_hi,
              acc, srcb, dstb, b0, b1, s0, s1):
    cid = lax.axis_index("c")
    sid = lax.axis_index("s")
    bufs = (b0, b1)
    sems = (s0, s1)

    def one_relation(h_half, src4, dst4, out_half):
        pltpu.sync_copy(zeros_h, acc.at[pl.ds(sid * RPT, RPT)])
        plsc.subcore_barrier()

        for st in range(NSTG):
            pltpu.sync_copy(src4.at[sid, st], srcb)
            pltpu.sync_copy(dst4.at[sid, st], dstb)
            for b in range(NB):
                pltpu.async_copy(h_half.at[srcb.at[b]], bufs[b], sems[b])

            def grp(g, carry):
                for b in range(NB):
                    jj = g * NB + b
                    pltpu.make_async_copy(
                        h_half.at[srcb.at[jj]], bufs[b], sems[b]).wait()
                    pltpu.sync_copy(bufs[b], acc.at[dstb.at[jj]], add=True)
                    pltpu.async_copy(
                        h_half.at[srcb.at[jj + NB]], bufs[b], sems[b])
                return carry

            lax.fori_loop(0, (S - NB) // NB, grp, 0)
            for b in range(NB):
                jj = S - NB + b
                pltpu.make_async_copy(
                    h_half.at[srcb.at[jj]], bufs[b], sems[b]).wait()
                pltpu.sync_copy(bufs[b], acc.at[dstb.at[jj]], add=True)

        plsc.subcore_barrier()
        pltpu.sync_copy(acc.at[pl.ds(sid * RPT, RPT)],
                        out_half.at[pl.ds(sid * RPT, RPT)])
        plsc.subcore_barrier()

    @pl.when(cid == 0)
    def _():
        one_relation(h_lo, src4_t, dst4_t, st_lo)
        one_relation(h_lo, src4_c, dst4_c, sc_lo)

    @pl.when(cid == 1)
    def _():
        one_relation(h_hi, src4_t, dst4_t, st_hi)
        one_relation(h_hi, src4_c, dst4_c, sc_hi)


_agg_call = pl.kernel(
    _agg_body,
    out_type=[jax.ShapeDtypeStruct((NP, HD), jnp.float32)] * 4,
    mesh=plsc.VectorSubcoreMesh(core_axis_name="c", subcore_axis_name="s"),
    scratch_types=(
        [pltpu.VMEM_SHARED((NP, HD), jnp.float32),
         pltpu.VMEM((S, K), jnp.int32),
         pltpu.VMEM((S, K), jnp.int32)]
        + [pltpu.VMEM((K, HD), jnp.float32) for _ in range(NB)]
        + [pltpu.SemaphoreType.DMA for _ in range(NB)]
    ),
)


def _cnt_body(dst4_t, dst4_c, zeros_h, ones_h, cnt_t, cnt_c,
              acc, dstb, ones_v, sem):
    cid = lax.axis_index("c")
    sid = lax.axis_index("s")

    def run(dst4, out):
        pltpu.sync_copy(zeros_h, acc.at[pl.ds(sid * RPT, RPT)])
        pltpu.sync_copy(ones_h, ones_v)
        plsc.subcore_barrier()

        for st in range(NSTG):
            pltpu.sync_copy(dst4.at[sid, st], dstb)

            def grp(g, carry):
                for b in range(NB):
                    pltpu.async_copy(
                        ones_v, acc.at[dstb.at[g * NB + b]], sem, add=True)
                for b in range(NB):
                    pltpu.make_async_copy(
                        ones_v, acc.at[dstb.at[g * NB + b]], sem).wait()
                return carry

            lax.fori_loop(0, S // NB, grp, 0)

        plsc.subcore_barrier()
        pltpu.sync_copy(acc.at[pl.ds(sid * RPT, RPT)],
                        out.at[pl.ds(sid * RPT, RPT)])

    @pl.when(cid == 0)
    def _():
        run(dst4_t, cnt_t)

    @pl.when(cid == 1)
    def _():
        run(dst4_c, cnt_c)


_cnt_call = pl.kernel(
    _cnt_body,
    out_type=[jax.ShapeDtypeStruct((NP, HD), jnp.float32)] * 2,
    mesh=plsc.VectorSubcoreMesh(core_axis_name="c", subcore_axis_name="s"),
    scratch_types=[
        pltpu.VMEM_SHARED((NP, HD), jnp.float32),
        pltpu.VMEM((S, K), jnp.int32),
        pltpu.VMEM((K, HD), jnp.float32),
        pltpu.SemaphoreType.DMA,
    ],
)


def _ln_rows(o, g, b):
    m = jnp.mean(o, axis=-1, keepdims=True)
    v = jnp.mean((o - m) * (o - m), axis=-1, keepdims=True)
    return (o - m) * lax.rsqrt(v + 1e-5) * g + b


def _in_body(x_ref, w_ref, b_ref, g_ref, be_ref, lo_ref, hi_ref):
    o = jnp.dot(x_ref[...], w_ref[...],
                preferred_element_type=jnp.float32) + b_ref[...]
    h = jnp.maximum(_ln_rows(o, g_ref[...], be_ref[...]), 0.0)
    lo_ref[...] = h[:, :HD]
    hi_ref[...] = h[:, HD:]


_in_call = pl.pallas_call(
    _in_body,
    grid=(GRID,),
    in_specs=[
        pl.BlockSpec((RB, D), lambda i: (i, 0)),
        pl.BlockSpec((D, D), lambda i: (0, 0)),
        pl.BlockSpec((1, D), lambda i: (0, 0)),
        pl.BlockSpec((1, D), lambda i: (0, 0)),
        pl.BlockSpec((1, D), lambda i: (0, 0)),
    ],
    out_specs=[pl.BlockSpec((RB, HD), lambda i: (i, 0))] * 2,
    out_shape=[jax.ShapeDtypeStruct((N, HD), jnp.float32)] * 2,
    compiler_params=pltpu.CompilerParams(
        dimension_semantics=("parallel",)),
)


def _comb_body(hlo, hhi, stlo, sthi, sclo, schi, ct, cc,
               wlt, wlc, wrt, wrc, blt, blc, g, b, olo, ohi):
    rt = 1.0 / jnp.maximum(ct[...], 1.0)
    rc = 1.0 / jnp.maximum(cc[...], 1.0)
    at = jnp.concatenate([stlo[...], sthi[...]], axis=1) * rt
    ac = jnp.concatenate([sclo[...], schi[...]], axis=1) * rc
    h = jnp.concatenate([hlo[...], hhi[...]], axis=1)
    o = (jnp.dot(at, wlt[...], preferred_element_type=jnp.float32)
         + jnp.dot(ac, wlc[...], preferred_element_type=jnp.float32)
         + jnp.dot(h, wrt[...] + wrc[...],
                   preferred_element_type=jnp.float32)
         + blt[...] + blc[...] + h)
    hn = _ln_rows(o, g[...], b[...])
    olo[...] = hn[:, :HD]
    ohi[...] = hn[:, HD:]


_comb_call = pl.pallas_call(
    _comb_body,
    grid=(GRID,),
    in_specs=(
        [pl.BlockSpec((RB, HD), lambda i: (i, 0))] * 6
        + [pl.BlockSpec((RB, 1), lambda i: (i, 0))] * 2
        + [pl.BlockSpec((D, D), lambda i: (0, 0))] * 4
        + [pl.BlockSpec((1, D), lambda i: (0, 0))] * 4
    ),
    out_specs=[pl.BlockSpec((RB, HD), lambda i: (i, 0))] * 2,
    out_shape=[jax.ShapeDtypeStruct((N, HD), jnp.float32)] * 2,
    compiler_params=pltpu.CompilerParams(
        dimension_semantics=("parallel",)),
)


def _comb_heads_body(hlo, hhi, stlo, sthi, sclo, schi, ct, cc,
                     wlt, wlc, wrt, wrc, blt, blc, g, b,
                     wg1, bg1, wg2, bg2, wd1, bd1, wd2, bd2,
                     gate_ref, delay_ref):
    rt = 1.0 / jnp.maximum(ct[...], 1.0)
    rc = 1.0 / jnp.maximum(cc[...], 1.0)
    at = jnp.concatenate([stlo[...], sthi[...]], axis=1) * rt
    ac = jnp.concatenate([sclo[...], schi[...]], axis=1) * rc
    h = jnp.concatenate([hlo[...], hhi[...]], axis=1)
    o = (jnp.dot(at, wlt[...], preferred_element_type=jnp.float32)
         + jnp.dot(ac, wlc[...], preferred_element_type=jnp.float32)
         + jnp.dot(h, wrt[...] + wrc[...],
                   preferred_element_type=jnp.float32)
         + blt[...] + blc[...] + h)
    hn = _ln_rows(o, g[...], b[...])
    tg = jnp.maximum(
        jnp.dot(hn, wg1[...], preferred_element_type=jnp.float32)
        + bg1[...], 0.0)
    gate_ref[...] = jnp.dot(
        tg, wg2[...], preferred_element_type=jnp.float32) + bg2[...]
    td = jnp.maximum(
        jnp.dot(hn, wd1[...], preferred_element_type=jnp.float32)
        + bd1[...], 0.0)
    delay_ref[...] = jnp.dot(
        td, wd2[...], preferred_element_type=jnp.float32) + bd2[...]


_comb_heads_call = pl.pallas_call(
    _comb_heads_body,
    grid=(GRID,),
    in_specs=(
        [pl.BlockSpec((RB, HD), lambda i: (i, 0))] * 6
        + [pl.BlockSpec((RB, 1), lambda i: (i, 0))] * 2
        + [pl.BlockSpec((D, D), lambda i: (0, 0))] * 4
        + [pl.BlockSpec((1, D), lambda i: (0, 0))] * 4
        + [pl.BlockSpec((D, HD), lambda i: (0, 0)),
           pl.BlockSpec((1, HD), lambda i: (0, 0)),
           pl.BlockSpec((HD, HD), lambda i: (0, 0)),
           pl.BlockSpec((1, HD), lambda i: (0, 0)),
           pl.BlockSpec((D, HD), lambda i: (0, 0)),
           pl.BlockSpec((1, HD), lambda i: (0, 0)),
           pl.BlockSpec((HD, HD), lambda i: (0, 0)),
           pl.BlockSpec((1, HD), lambda i: (0, 0))]
    ),
    out_specs=[pl.BlockSpec((RB, HD), lambda i: (i, 0))] * 2,
    out_shape=[jax.ShapeDtypeStruct((N, HD), jnp.float32)] * 2,
    compiler_params=pltpu.CompilerParams(
        dimension_semantics=("parallel",)),
)


def _edge_views(ei):
    ei = ei.astype(jnp.int32)
    pad = jnp.arange(EP - E, dtype=jnp.int32)
    src = jnp.concatenate([ei[0], pad % N])
    dst = jnp.concatenate([ei[1], N + pad % (NP - N)])
    return (src.reshape(NS, NSTG, S, K), dst.reshape(NS, NSTG, S, K))


def kernel(x, W_in, b_in, g_in, be_in, Wl_t, bl_t, Wr_t, Wl_c, bl_c, Wr_c,
           g_ln, b_ln, Wg1, bg1, Wg2, bg2, Wd1, bd1, Wd2, bd2, ei_t, ei_c):
    f32 = jnp.float32
    src4_t, dst4_t = _edge_views(ei_t)
    src4_c, dst4_c = _edge_views(ei_c)
    zeros_h = jnp.zeros((RPT, HD), f32)
    ones_h = jnp.ones((K, HD), f32)

    cnt_t_full, cnt_c_full = _cnt_call(dst4_t, dst4_c, zeros_h, ones_h)
    cnt_t = cnt_t_full[:, :1]
    cnt_c = cnt_c_full[:, :1]
    h_lo, h_hi = _in_call(x, W_in, b_in.reshape(1, D), g_in.reshape(1, D),
                          be_in.reshape(1, D))

    for l in range(L - 1):
        st_lo, st_hi, sc_lo, sc_hi = _agg_call(
            h_lo, h_hi, src4_t, dst4_t, src4_c, dst4_c, zeros_h)
        h_lo, h_hi = _comb_call(
            h_lo, h_hi, st_lo, st_hi, sc_lo, sc_hi, cnt_t, cnt_c,
            Wl_t[l], Wl_c[l], Wr_t[l], Wr_c[l],
            bl_t[l].reshape(1, D), bl_c[l].reshape(1, D),
            g_ln[l].reshape(1, D), b_ln[l].reshape(1, D))

    wg2p = jnp.pad(Wg2, ((0, 0), (0, HD - NG)))
    bg2p = jnp.pad(bg2, (0, HD - NG)).reshape(1, HD)
    wd2p = jnp.pad(Wd2, ((0, 0), (0, HD - 1)))
    bd2p = jnp.pad(bd2, (0, HD - 1)).reshape(1, HD)
    l = L - 1
    st_lo, st_hi, sc_lo, sc_hi = _agg_call(
        h_lo, h_hi, src4_t, dst4_t, src4_c, dst4_c, zeros_h)
    gate_p, delay_p = _comb_heads_call(
        h_lo, h_hi, st_lo, st_hi, sc_lo, sc_hi, cnt_t, cnt_c,
        Wl_t[l], Wl_c[l], Wr_t[l], Wr_c[l],
        bl_t[l].reshape(1, D), bl_c[l].reshape(1, D),
        g_ln[l].reshape(1, D), b_ln[l].reshape(1, D),
        Wg1, bg1.reshape(1, HD), wg2p, bg2p,
        Wd1, bd1.reshape(1, HD), wd2p, bd2p)
    return gate_p[:, :NG], delay_p[:, 0]


# speedup vs baseline: 5.7631x; 1.0025x over previous
import jax
import jax.numpy as jnp
from jax import lax
from jax.experimental import pallas as pl
from jax.experimental.pallas import tpu as pltpu
from jax.experimental.pallas import tpu_sc as plsc

N = 10000
E = 160000
D = 256
HD = 128
NG = 5
L = 3

NS = 16
K = 128
S = 20
NSTG = 4
EPT = K * S * NSTG
EP = NS * EPT
NB = 2
NP = 10112
RPT = NP // NS

RB = 400
GRID = N // RB


def _agg_body(h_lo, h_hi, src4_t, dst4_t, src4_c, dst4_c, zeros_h,
              st_lo, st_hi, sc_lo, sc_hi,
              acc, srcb, dstb, b0, b1, s0, s1):
    cid = lax.axis_index("c")
    sid = lax.axis_index("s")
    bufs = (b0, b1)
    sems = (s0, s1)

    def one_relation(h_half, src4, dst4, out_half):
        pltpu.sync_copy(zeros_h, acc.at[pl.ds(sid * RPT, RPT)])
        plsc.subcore_barrier()

        for st in range(NSTG):
            pltpu.sync_copy(src4.at[sid, st], srcb)
            pltpu.sync_copy(dst4.at[sid, st], dstb)
            for b in range(NB):
                pltpu.async_copy(h_half.at[srcb.at[b]], bufs[b], sems[b])

            def grp(g, carry):
                for b in range(NB):
                    jj = g * NB + b
                    pltpu.make_async_copy(
                        h_half.at[srcb.at[jj]], bufs[b], sems[b]).wait()
                    pltpu.sync_copy(bufs[b], acc.at[dstb.at[jj]], add=True)
                    pltpu.async_copy(
                        h_half.at[srcb.at[jj + NB]], bufs[b], sems[b])
                return carry

            lax.fori_loop(0, (S - NB) // NB, grp, 0)
            for b in range(NB):
                jj = S - NB + b
                pltpu.make_async_copy(
                    h_half.at[srcb.at[jj]], bufs[b], sems[b]).wait()
                pltpu.sync_copy(bufs[b], acc.at[dstb.at[jj]], add=True)

        plsc.subcore_barrier()
        pltpu.sync_copy(acc.at[pl.ds(sid * RPT, RPT)],
                        out_half.at[pl.ds(sid * RPT, RPT)])
        plsc.subcore_barrier()

    @pl.when(cid == 0)
    def _():
        one_relation(h_lo, src4_t, dst4_t, st_lo)
        one_relation(h_lo, src4_c, dst4_c, sc_lo)

    @pl.when(cid == 1)
    def _():
        one_relation(h_hi, src4_t, dst4_t, st_hi)
        one_relation(h_hi, src4_c, dst4_c, sc_hi)


_agg_call = pl.kernel(
    _agg_body,
    out_type=[jax.ShapeDtypeStruct((NP, HD), jnp.float32)] * 4,
    mesh=plsc.VectorSubcoreMesh(core_axis_name="c", subcore_axis_name="s"),
    scratch_types=(
        [pltpu.VMEM_SHARED((NP, HD), jnp.float32),
         pltpu.VMEM((S, K), jnp.int32),
         pltpu.VMEM((S, K), jnp.int32)]
        + [pltpu.VMEM((K, HD), jnp.float32) for _ in range(NB)]
        + [pltpu.SemaphoreType.DMA for _ in range(NB)]
    ),
)


def _cnt_body(dst4_t, dst4_c, zeros_h, ones_h, cnt_t, cnt_c,
              acc, dstb, onesb):
    cid = lax.axis_index("c")
    sid = lax.axis_index("s")

    def run(dst4, out):
        pltpu.sync_copy(zeros_h, acc.at[pl.ds(sid * RPT, RPT)])
        pltpu.sync_copy(ones_h, onesb)
        plsc.subcore_barrier()

        for st in range(NSTG):
            pltpu.sync_copy(dst4.at[sid, st], dstb)

            def body(j, c):
                pltpu.sync_copy(onesb, acc.at[dstb.at[j]], add=True)
                return c

            lax.fori_loop(0, S, body, 0)

        plsc.subcore_barrier()
        pltpu.sync_copy(acc.at[pl.ds(sid * RPT, RPT)],
                        out.at[pl.ds(sid * RPT, RPT)])
        plsc.subcore_barrier()

    @pl.when(cid == 0)
    def _():
        run(dst4_t, cnt_t)

    @pl.when(cid == 1)
    def _():
        run(dst4_c, cnt_c)


_cnt_call = pl.kernel(
    _cnt_body,
    out_type=[jax.ShapeDtypeStruct((NP, HD), jnp.float32)] * 2,
    mesh=plsc.VectorSubcoreMesh(core_axis_name="c", subcore_axis_name="s"),
    scratch_types=[
        pltpu.VMEM_SHARED((NP, HD), jnp.float32),
        pltpu.VMEM((S, K), jnp.int32),
        pltpu.VMEM((K, HD), jnp.float32),
    ],
)


def _ln_rows(o, g, b):
    m = jnp.mean(o, axis=-1, keepdims=True)
    v = jnp.mean((o - m) * (o - m), axis=-1, keepdims=True)
    return (o - m) * lax.rsqrt(v + 1e-5) * g + b


def _in_body(x_ref, w_ref, b_ref, g_ref, be_ref, lo_ref, hi_ref):
    o = jnp.dot(x_ref[...], w_ref[...],
                preferred_element_type=jnp.float32) + b_ref[...]
    h = jnp.maximum(_ln_rows(o, g_ref[...], be_ref[...]), 0.0)
    lo_ref[...] = h[:, :HD]
    hi_ref[...] = h[:, HD:]


_in_call = pl.pallas_call(
    _in_body,
    grid=(GRID,),
    in_specs=[
        pl.BlockSpec((RB, D), lambda i: (i, 0)),
        pl.BlockSpec((D, D), lambda i: (0, 0)),
        pl.BlockSpec((1, D), lambda i: (0, 0)),
        pl.BlockSpec((1, D), lambda i: (0, 0)),
        pl.BlockSpec((1, D), lambda i: (0, 0)),
    ],
    out_specs=[pl.BlockSpec((RB, HD), lambda i: (i, 0))] * 2,
    out_shape=[jax.ShapeDtypeStruct((N, HD), jnp.float32)] * 2,
    compiler_params=pltpu.CompilerParams(
        dimension_semantics=("parallel",)),
)


def _comb_body(hlo, hhi, stlo, sthi, sclo, schi, ct, cc,
               wlt, wlc, wrt, wrc, blt, blc, g, b, olo, ohi):
    rt = 1.0 / jnp.maximum(ct[...], 1.0)
    rc = 1.0 / jnp.maximum(cc[...], 1.0)
    at = jnp.concatenate([stlo[...], sthi[...]], axis=1) * rt
    ac = jnp.concatenate([sclo[...], schi[...]], axis=1) * rc
    h = jnp.concatenate([hlo[...], hhi[...]], axis=1)
    o = (jnp.dot(at, wlt[...], preferred_element_type=jnp.float32)
         + jnp.dot(ac, wlc[...], preferred_element_type=jnp.float32)
         + jnp.dot(h, wrt[...] + wrc[...],
                   preferred_element_type=jnp.float32)
         + blt[...] + blc[...] + h)
    hn = _ln_rows(o, g[...], b[...])
    olo[...] = hn[:, :HD]
    ohi[...] = hn[:, HD:]


_comb_call = pl.pallas_call(
    _comb_body,
    grid=(GRID,),
    in_specs=(
        [pl.BlockSpec((RB, HD), lambda i: (i, 0))] * 6
        + [pl.BlockSpec((RB, 1), lambda i: (i, 0))] * 2
        + [pl.BlockSpec((D, D), lambda i: (0, 0))] * 4
        + [pl.BlockSpec((1, D), lambda i: (0, 0))] * 4
    ),
    out_specs=[pl.BlockSpec((RB, HD), lambda i: (i, 0))] * 2,
    out_shape=[jax.ShapeDtypeStruct((N, HD), jnp.float32)] * 2,
    compiler_params=pltpu.CompilerParams(
        dimension_semantics=("parallel",)),
)


def _comb_heads_body(hlo, hhi, stlo, sthi, sclo, schi, ct, cc,
                     wlt, wlc, wrt, wrc, blt, blc, g, b,
                     wg1, bg1, wg2, bg2, wd1, bd1, wd2, bd2,
                     gate_ref, delay_ref):
    rt = 1.0 / jnp.maximum(ct[...], 1.0)
    rc = 1.0 / jnp.maximum(cc[...], 1.0)
    at = jnp.concatenate([stlo[...], sthi[...]], axis=1) * rt
    ac = jnp.concatenate([sclo[...], schi[...]], axis=1) * rc
    h = jnp.concatenate([hlo[...], hhi[...]], axis=1)
    o = (jnp.dot(at, wlt[...], preferred_element_type=jnp.float32)
         + jnp.dot(ac, wlc[...], preferred_element_type=jnp.float32)
         + jnp.dot(h, wrt[...] + wrc[...],
                   preferred_element_type=jnp.float32)
         + blt[...] + blc[...] + h)
    hn = _ln_rows(o, g[...], b[...])
    tg = jnp.maximum(
        jnp.dot(hn, wg1[...], preferred_element_type=jnp.float32)
        + bg1[...], 0.0)
    gate_ref[...] = jnp.dot(
        tg, wg2[...], preferred_element_type=jnp.float32) + bg2[...]
    td = jnp.maximum(
        jnp.dot(hn, wd1[...], preferred_element_type=jnp.float32)
        + bd1[...], 0.0)
    delay_ref[...] = jnp.dot(
        td, wd2[...], preferred_element_type=jnp.float32) + bd2[...]


_comb_heads_call = pl.pallas_call(
    _comb_heads_body,
    grid=(GRID,),
    in_specs=(
        [pl.BlockSpec((RB, HD), lambda i: (i, 0))] * 6
        + [pl.BlockSpec((RB, 1), lambda i: (i, 0))] * 2
        + [pl.BlockSpec((D, D), lambda i: (0, 0))] * 4
        + [pl.BlockSpec((1, D), lambda i: (0, 0))] * 4
        + [pl.BlockSpec((D, HD), lambda i: (0, 0)),
           pl.BlockSpec((1, HD), lambda i: (0, 0)),
           pl.BlockSpec((HD, HD), lambda i: (0, 0)),
           pl.BlockSpec((1, HD), lambda i: (0, 0)),
           pl.BlockSpec((D, HD), lambda i: (0, 0)),
           pl.BlockSpec((1, HD), lambda i: (0, 0)),
           pl.BlockSpec((HD, HD), lambda i: (0, 0)),
           pl.BlockSpec((1, HD), lambda i: (0, 0))]
    ),
    out_specs=[pl.BlockSpec((RB, HD), lambda i: (i, 0))] * 2,
    out_shape=[jax.ShapeDtypeStruct((N, HD), jnp.float32)] * 2,
    compiler_params=pltpu.CompilerParams(
        dimension_semantics=("parallel",)),
)


def _edge_views(ei):
    ei = ei.astype(jnp.int32)
    pad = jnp.arange(EP - E, dtype=jnp.int32)
    src = jnp.concatenate([ei[0], pad % N])
    dst = jnp.concatenate([ei[1], N + pad % (NP - N)])
    return (src.reshape(NS, NSTG, S, K), dst.reshape(NS, NSTG, S, K))


def kernel(x, W_in, b_in, g_in, be_in, Wl_t, bl_t, Wr_t, Wl_c, bl_c, Wr_c,
           g_ln, b_ln, Wg1, bg1, Wg2, bg2, Wd1, bd1, Wd2, bd2, ei_t, ei_c):
    f32 = jnp.float32
    src4_t, dst4_t = _edge_views(ei_t)
    src4_c, dst4_c = _edge_views(ei_c)
    zeros_h = jnp.zeros((RPT, HD), f32)
    ones_h = jnp.ones((K, HD), f32)

    cnt_t_full, cnt_c_full = _cnt_call(dst4_t, dst4_c, zeros_h, ones_h)
    cnt_t = cnt_t_full[:, :1]
    cnt_c = cnt_c_full[:, :1]
    h_lo, h_hi = _in_call(x, W_in, b_in.reshape(1, D), g_in.reshape(1, D),
                          be_in.reshape(1, D))

    for l in range(L - 1):
        st_lo, st_hi, sc_lo, sc_hi = _agg_call(
            h_lo, h_hi, src4_t, dst4_t, src4_c, dst4_c, zeros_h)
        h_lo, h_hi = _comb_call(
            h_lo, h_hi, st_lo, st_hi, sc_lo, sc_hi, cnt_t, cnt_c,
            Wl_t[l], Wl_c[l], Wr_t[l], Wr_c[l],
            bl_t[l].reshape(1, D), bl_c[l].reshape(1, D),
            g_ln[l].reshape(1, D), b_ln[l].reshape(1, D))

    wg2p = jnp.pad(Wg2, ((0, 0), (0, HD - NG)))
    bg2p = jnp.pad(bg2, (0, HD - NG)).reshape(1, HD)
    wd2p = jnp.pad(Wd2, ((0, 0), (0, HD - 1)))
    bd2p = jnp.pad(bd2, (0, HD - 1)).reshape(1, HD)
    l = L - 1
    st_lo, st_hi, sc_lo, sc_hi = _agg_call(
        h_lo, h_hi, src4_t, dst4_t, src4_c, dst4_c, zeros_h)
    gate_p, delay_p = _comb_heads_call(
        h_lo, h_hi, st_lo, st_hi, sc_lo, sc_hi, cnt_t, cnt_c,
        Wl_t[l], Wl_c[l], Wr_t[l], Wr_c[l],
        bl_t[l].reshape(1, D), bl_c[l].reshape(1, D),
        g_ln[l].reshape(1, D), b_ln[l].reshape(1, D),
        Wg1, bg1.reshape(1, HD), wg2p, bg2p,
        Wd1, bd1.reshape(1, HD), wd2p, bd2p)
    return gate_p[:, :NG], delay_p[:, 0]
